# baseline (speedup 1.0000x reference)
import functools

import jax
import jax.numpy as jnp
from jax import lax
from jax.experimental import pallas as pl
from jax.experimental.pallas import tpu as pltpu
from jax.experimental.pallas import tpu_sc as plsc

B, S, DE, HIDDEN = 8, 1024, 4, 256
BASE_CHUNK = 64
MAX_SEQ_LEN = 512

NB = 3
NW = 32
ROWS_SC = NB * S
RPW = ROWS_SC // NW
CH = 8
STAGES = RPW // CH
SLAB = DE * S
TS2 = 512
K2 = S // TS2


def _finalize(acc_mean_sum):
    cs = acc_mean_sum * (float(BASE_CHUNK) / float(S))
    cs = jnp.clip(cs, 32.0, 128.0)
    bad = (cs < 1.0) | ~jnp.isfinite(cs)
    return jnp.where(bad, float(BASE_CHUNK), cs)


def _mlp_tail(es, node, mask_row, wn_ref, bn_ref, we1_ref, be1_ref, we2_ref,
              be2_ref, wia_ref, wib_ref, bi_ref):
    node_enc = (jnp.dot(node, wn_ref[...],
                        preferred_element_type=jnp.float32)
                + bn_ref[...])
    h = jnp.maximum(es * we1_ref[...] + be1_ref[...], 0.0)
    edge_enc = (jnp.dot(h, we2_ref[...],
                        preferred_element_type=jnp.float32)
                + be2_ref[...])
    logit = (jnp.sum(node_enc * wia_ref[...], axis=1, keepdims=True)
             + jnp.sum(edge_enc * wib_ref[...], axis=1, keepdims=True)
             + bi_ref[0, 0])
    sig = jax.nn.sigmoid(logit)
    acc = jnp.dot(mask_row, sig, preferred_element_type=jnp.float32)
    return acc[0, 0]


def _tc_kernel(edge_ref, node_ref, mask_ref, wn_ref, bn_ref, we1_ref,
               be1_ref, we2_ref, be2_ref, wia_ref, wib_ref, bi_ref,
               out_ref):
    k = pl.program_id(1)
    e = edge_ref[0]
    r1 = jnp.sum(e, axis=2)
    rowsum = jnp.sum(r1, axis=1, keepdims=True)
    es = rowsum * (1.0 / (S * DE))
    acc = _mlp_tail(es, node_ref[0], mask_ref[0, 0], wn_ref, bn_ref,
                    we1_ref, be1_ref, we2_ref, be2_ref, wia_ref, wib_ref,
                    bi_ref)

    @pl.when(k == 0)
    def _init():
        out_ref[...] = jnp.zeros_like(out_ref)

    out_ref[...] += acc

    @pl.when(k == K2 - 1)
    def _fin():
        out_ref[...] = _finalize(out_ref[...])


def _tc_tail_kernel(rs_ref, node_ref, mask_ref, wn_ref, bn_ref, we1_ref,
                    be1_ref, we2_ref, be2_ref, wia_ref, wib_ref, bi_ref,
                    out_ref):
    es = (jnp.sum(rs_ref[0], axis=1, keepdims=True)
          * (1.0 / (S * DE)))
    acc = _mlp_tail(es, node_ref[0], mask_ref[0, 0], wn_ref, bn_ref,
                    we1_ref, be1_ref, we2_ref, be2_ref, wia_ref, wib_ref,
                    bi_ref)
    out_ref[...] = jnp.full_like(out_ref, _finalize(acc))


def _sc_rowsums(edge_hbm, out_hbm, buf0, buf1, sums, sem0, sem1):
    wid = lax.axis_index("s") * 2 + lax.axis_index("c")
    base = wid * RPW
    bufs = (buf0, buf1)
    sems = (sem0, sem1)
    handles = {}
    handles[0] = pltpu.async_copy(
        edge_hbm.at[pl.ds(base, CH)], bufs[0], sems[0])
    for g in range(STAGES):
        if g + 1 < STAGES:
            handles[g + 1] = pltpu.async_copy(
                edge_hbm.at[pl.ds(base + (g + 1) * CH, CH)],
                bufs[(g + 1) % 2], sems[(g + 1) % 2])
        handles[g].wait()
        buf = bufs[g % 2]
        for c in range(CH):
            zero = jnp.zeros((16,), jnp.float32)

            def body(i, accs, _c=c):
                a0, a1, a2, a3 = accs
                a0 = a0 + buf[_c, 0, pl.ds(i * 16, 16)]
                a1 = a1 + buf[_c, 1, pl.ds(i * 16, 16)]
                a2 = a2 + buf[_c, 2, pl.ds(i * 16, 16)]
                a3 = a3 + buf[_c, 3, pl.ds(i * 16, 16)]
                return (a0, a1, a2, a3)

            a0, a1, a2, a3 = lax.fori_loop(
                0, S // 16, body, (zero, zero, zero, zero))
            r = g * CH + c
            sums[pl.ds(r * 16, 16)] = (a0 + a1) + (a2 + a3)
    pltpu.sync_copy(sums, out_hbm.at[pl.ds(base * 16, RPW * 16)])


def kernel(node_features, edge_features, sequence_mask, W_node, b_node,
           W_e1, b_e1, W_e2, b_e2, W_imp, b_imp):
    edge_r = jnp.transpose(edge_features, (0, 1, 3, 2))
    edge_flat = edge_r.reshape(B * S, DE, S)
    mask_r = sequence_mask.reshape(B, 1, 1, S)
    bn = b_node.reshape(1, 32)
    be1 = b_e1.reshape(1, 64)
    be2 = b_e2.reshape(1, 32)
    wia = W_imp[:32].reshape(1, 32)
    wib = W_imp[32:].reshape(1, 32)
    bi = b_imp.reshape(1, 1)
    weights = (W_node, bn, W_e1, be1, W_e2, be2, wia, wib, bi)
    weight_specs = [
        pl.BlockSpec((HIDDEN, 32), lambda b: (0, 0)),
        pl.BlockSpec((1, 32), lambda b: (0, 0)),
        pl.BlockSpec((1, 64), lambda b: (0, 0)),
        pl.BlockSpec((1, 64), lambda b: (0, 0)),
        pl.BlockSpec((64, 32), lambda b: (0, 0)),
        pl.BlockSpec((1, 32), lambda b: (0, 0)),
        pl.BlockSpec((1, 32), lambda b: (0, 0)),
        pl.BlockSpec((1, 32), lambda b: (0, 0)),
        pl.BlockSpec((1, 1), lambda b: (0, 0)),
    ]

    mesh = plsc.VectorSubcoreMesh(core_axis_name="c", subcore_axis_name="s")
    sc_call = functools.partial(
        pl.kernel, _sc_rowsums, mesh=mesh,
        out_type=jax.ShapeDtypeStruct((ROWS_SC * 16,), jnp.float32),
        scratch_types=[
            pltpu.VMEM((CH, DE, S), jnp.float32),
            pltpu.VMEM((CH, DE, S), jnp.float32),
            pltpu.VMEM((RPW * 16,), jnp.float32),
            pltpu.SemaphoreType.DMA,
            pltpu.SemaphoreType.DMA,
        ],
    )()
    rowsums = sc_call(edge_flat)

    mask_k = sequence_mask.reshape(B, K2, 1, TS2)
    out_tc = pl.pallas_call(
        _tc_kernel,
        grid=(B - NB, K2),
        in_specs=[
            pl.BlockSpec((1, TS2, DE, S), lambda b, k: (b + NB, k, 0, 0)),
            pl.BlockSpec((1, TS2, HIDDEN), lambda b, k: (b + NB, k, 0)),
            pl.BlockSpec((1, 1, 1, TS2), lambda b, k: (b + NB, k, 0, 0)),
        ] + [pl.BlockSpec(s.block_shape, (lambda f: (lambda b, k: f(b)))(s.index_map))
             for s in weight_specs],
        out_specs=pl.BlockSpec((1, 1, 128), lambda b, k: (b, 0, 0)),
        out_shape=jax.ShapeDtypeStruct((B - NB, 1, 128), jnp.float32),
        compiler_params=pltpu.CompilerParams(
            dimension_semantics=("arbitrary", "arbitrary")),
    )(edge_r, node_features, mask_k, *weights)

    rs3 = rowsums.reshape(NB, S, 16)
    out_sc = pl.pallas_call(
        _tc_tail_kernel,
        grid=(NB,),
        in_specs=[
            pl.BlockSpec((1, S, 16), lambda b: (b, 0, 0)),
            pl.BlockSpec((1, S, HIDDEN), lambda b: (b, 0, 0)),
            pl.BlockSpec((1, 1, 1, S), lambda b: (b, 0, 0, 0)),
        ] + weight_specs,
        out_specs=pl.BlockSpec((1, 1, 128), lambda b: (b, 0, 0)),
        out_shape=jax.ShapeDtypeStruct((NB, 1, 128), jnp.float32),
        compiler_params=pltpu.CompilerParams(
            dimension_semantics=("arbitrary",)),
    )(rs3, node_features, mask_r, *weights)

    chunk_sizes = jnp.concatenate([out_sc[:, 0, 0], out_tc[:, 0, 0]], axis=0)
    return (chunk_sizes, MAX_SEQ_LEN)


# speedup vs baseline: 1.3896x; 1.3896x over previous
import jax
import jax.numpy as jnp
from jax.experimental import pallas as pl
from jax.experimental.pallas import tpu as pltpu

B, S, DE, HIDDEN = 8, 1024, 4, 256
BASE_CHUNK = 64
MAX_SEQ_LEN = 512
TS = 1024
K = S // TS


def _chunk_kernel(edge_ref, node_ref, mask_ref, wn_ref, bn_ref, we1_ref,
                  be1_ref, we2_ref, be2_ref, wia_ref, wib_ref, bi_ref,
                  out_ref):
    k = pl.program_id(1)

    e = edge_ref[0]
    r1 = jnp.sum(e, axis=2)
    rowsum = jnp.sum(r1, axis=1, keepdims=True)
    es = rowsum * (1.0 / (S * DE))

    node_enc = (jnp.dot(node_ref[0], wn_ref[...],
                        preferred_element_type=jnp.float32)
                + bn_ref[...])
    h = jnp.maximum(es * we1_ref[...] + be1_ref[...], 0.0)
    edge_enc = (jnp.dot(h, we2_ref[...],
                        preferred_element_type=jnp.float32)
                + be2_ref[...])

    logit = (jnp.sum(node_enc * wia_ref[...], axis=1, keepdims=True)
             + jnp.sum(edge_enc * wib_ref[...], axis=1, keepdims=True)
             + bi_ref[0, 0])
    sig = jax.nn.sigmoid(logit)
    m = mask_ref[0, 0]
    partial = jnp.dot(m, sig, preferred_element_type=jnp.float32)

    @pl.when(k == 0)
    def _init():
        out_ref[...] = jnp.zeros_like(out_ref)

    out_ref[...] += partial[0, 0]

    @pl.when(k == K - 1)
    def _finalize():
        acc = out_ref[...]
        cs = acc * (float(BASE_CHUNK) / float(S))
        cs = jnp.clip(cs, 32.0, 128.0)
        bad = (cs < 1.0) | ~jnp.isfinite(cs)
        out_ref[...] = jnp.where(bad, float(BASE_CHUNK), cs)


def kernel(node_features, edge_features, sequence_mask, W_node, b_node,
           W_e1, b_e1, W_e2, b_e2, W_imp, b_imp):
    edge_r = jnp.transpose(edge_features, (0, 1, 3, 2))
    mask_r = sequence_mask.reshape(B, K, 1, TS)
    bn = b_node.reshape(1, 32)
    be1 = b_e1.reshape(1, 64)
    be2 = b_e2.reshape(1, 32)
    wia = W_imp[:32].reshape(1, 32)
    wib = W_imp[32:].reshape(1, 32)
    bi = b_imp.reshape(1, 1)

    out = pl.pallas_call(
        _chunk_kernel,
        grid=(B, K),
        in_specs=[
            pl.BlockSpec((1, TS, DE, S), lambda b, k: (b, k, 0, 0)),
            pl.BlockSpec((1, TS, HIDDEN), lambda b, k: (b, k, 0)),
            pl.BlockSpec((1, 1, 1, TS), lambda b, k: (b, k, 0, 0)),
            pl.BlockSpec((HIDDEN, 32), lambda b, k: (0, 0)),
            pl.BlockSpec((1, 32), lambda b, k: (0, 0)),
            pl.BlockSpec((1, 64), lambda b, k: (0, 0)),
            pl.BlockSpec((1, 64), lambda b, k: (0, 0)),
            pl.BlockSpec((64, 32), lambda b, k: (0, 0)),
            pl.BlockSpec((1, 32), lambda b, k: (0, 0)),
            pl.BlockSpec((1, 32), lambda b, k: (0, 0)),
            pl.BlockSpec((1, 32), lambda b, k: (0, 0)),
            pl.BlockSpec((1, 1), lambda b, k: (0, 0)),
        ],
        out_specs=pl.BlockSpec((1, 1, 128), lambda b, k: (b, 0, 0)),
        out_shape=jax.ShapeDtypeStruct((B, 1, 128), jnp.float32),
        compiler_params=pltpu.CompilerParams(
            dimension_semantics=("parallel", "arbitrary")),
    )(edge_r, node_features, mask_r, W_node, bn, W_e1, be1, W_e2, be2,
      wia, wib, bi)

    return (out[:, 0, 0], MAX_SEQ_LEN)


# speedup vs baseline: 1.4633x; 1.0530x over previous
import jax
import jax.numpy as jnp
from jax.experimental import pallas as pl
from jax.experimental.pallas import tpu as pltpu

B, S, DE, HIDDEN = 8, 1024, 4, 256
BASE_CHUNK = 64
MAX_SEQ_LEN = 512
TS = 1024
K = S // TS


def _chunk_kernel(edge_ref, node_ref, mask_ref, wn_ref, bn_ref, we1_ref,
                  be1_ref, we2_ref, be2_ref, wia_ref, wib_ref, bi_ref,
                  out_ref):
    k = pl.program_id(1)

    e = edge_ref[0]
    r1 = jnp.sum(e, axis=2)
    rowsum = jnp.sum(r1, axis=1, keepdims=True)
    es = rowsum * (1.0 / (S * DE))

    node_enc = (jax.lax.dot_general(
        node_ref[0], wn_ref[...], (((1,), (1,)), ((), ())),
        preferred_element_type=jnp.float32)
                + bn_ref[...])
    h = jnp.maximum(es * we1_ref[...] + be1_ref[...], 0.0)
    edge_enc = (jax.lax.dot_general(
        h, we2_ref[...], (((1,), (1,)), ((), ())),
        preferred_element_type=jnp.float32)
                + be2_ref[...])

    logit = (jnp.sum(node_enc * wia_ref[...], axis=1, keepdims=True)
             + jnp.sum(edge_enc * wib_ref[...], axis=1, keepdims=True)
             + bi_ref[0, 0])
    sig = jax.nn.sigmoid(logit)
    m = mask_ref[0, 0]
    partial = jnp.dot(m, sig, preferred_element_type=jnp.float32)

    @pl.when(k == 0)
    def _init():
        out_ref[...] = jnp.zeros_like(out_ref)

    out_ref[...] += partial[0, 0]

    @pl.when(k == K - 1)
    def _finalize():
        acc = out_ref[...]
        cs = acc * (float(BASE_CHUNK) / float(S))
        cs = jnp.clip(cs, 32.0, 128.0)
        bad = (cs < 1.0) | ~jnp.isfinite(cs)
        out_ref[...] = jnp.where(bad, float(BASE_CHUNK), cs)


def kernel(node_features, edge_features, sequence_mask, W_node, b_node,
           W_e1, b_e1, W_e2, b_e2, W_imp, b_imp):
    edge_r = jnp.transpose(edge_features, (0, 1, 3, 2))
    wn_t = jnp.transpose(W_node)
    we2_t = jnp.transpose(W_e2)
    mask_r = sequence_mask.reshape(B, K, 1, TS)
    bn = b_node.reshape(1, 32)
    be1 = b_e1.reshape(1, 64)
    be2 = b_e2.reshape(1, 32)
    wia = W_imp[:32].reshape(1, 32)
    wib = W_imp[32:].reshape(1, 32)
    bi = b_imp.reshape(1, 1)

    out = pl.pallas_call(
        _chunk_kernel,
        grid=(B, K),
        in_specs=[
            pl.BlockSpec((1, TS, DE, S), lambda b, k: (b, k, 0, 0)),
            pl.BlockSpec((1, TS, HIDDEN), lambda b, k: (b, k, 0)),
            pl.BlockSpec((1, 1, 1, TS), lambda b, k: (b, k, 0, 0)),
            pl.BlockSpec((32, HIDDEN), lambda b, k: (0, 0)),
            pl.BlockSpec((1, 32), lambda b, k: (0, 0)),
            pl.BlockSpec((1, 64), lambda b, k: (0, 0)),
            pl.BlockSpec((1, 64), lambda b, k: (0, 0)),
            pl.BlockSpec((32, 64), lambda b, k: (0, 0)),
            pl.BlockSpec((1, 32), lambda b, k: (0, 0)),
            pl.BlockSpec((1, 32), lambda b, k: (0, 0)),
            pl.BlockSpec((1, 32), lambda b, k: (0, 0)),
            pl.BlockSpec((1, 1), lambda b, k: (0, 0)),
        ],
        out_specs=pl.BlockSpec((1, 1, 128), lambda b, k: (b, 0, 0)),
        out_shape=jax.ShapeDtypeStruct((B, 1, 128), jnp.float32),
        compiler_params=pltpu.CompilerParams(
            dimension_semantics=("parallel", "arbitrary")),
    )(edge_r, node_features, mask_r, wn_t, bn, W_e1, be1, we2_t, be2,
      wia, wib, bi)

    return (out[:, 0, 0], MAX_SEQ_LEN)


# speedup vs baseline: 1.5434x; 1.0548x over previous
import jax
import jax.numpy as jnp
from jax.experimental import pallas as pl
from jax.experimental.pallas import tpu as pltpu

B, S, DE, HIDDEN = 8, 1024, 4, 256
BASE_CHUNK = 64
MAX_SEQ_LEN = 512
TS = 1024
K = S // TS


def _chunk_kernel(edge_ref, node_ref, mask_ref, wn_ref, bn_ref, we1_ref,
                  be1_ref, we2_ref, be2_ref, wimp_ref, bi_ref,
                  out_ref):
    b = pl.program_id(0)
    k = pl.program_id(1)

    e = edge_ref[0]
    r1 = jnp.sum(e, axis=2)
    rowsum = jnp.sum(r1, axis=1, keepdims=True)
    es = rowsum * (1.0 / (S * DE))

    node_enc = (jax.lax.dot_general(
        node_ref[0], wn_ref[...], (((1,), (1,)), ((), ())),
        preferred_element_type=jnp.float32)
                + bn_ref[...])
    h = jnp.maximum(es * we1_ref[...] + be1_ref[...], 0.0)
    edge_enc = (jax.lax.dot_general(
        h, we2_ref[...], (((1,), (1,)), ((), ())),
        preferred_element_type=jnp.float32)
                + be2_ref[...])

    logit = (jnp.sum(node_enc * wimp_ref[:, :32], axis=1, keepdims=True)
             + jnp.sum(edge_enc * wimp_ref[:, 32:], axis=1, keepdims=True)
             + bi_ref[0, 0])
    sig = jax.nn.sigmoid(logit)
    m = mask_ref[pl.ds(b, 1), pl.ds(k * TS, TS)]
    partial = jnp.dot(m, sig, preferred_element_type=jnp.float32)

    @pl.when(k == 0)
    def _init():
        out_ref[...] = jnp.zeros_like(out_ref)

    out_ref[...] += partial[0, 0]

    @pl.when(k == K - 1)
    def _finalize():
        acc = out_ref[...]
        cs = acc * (float(BASE_CHUNK) / float(S))
        cs = jnp.clip(cs, 32.0, 128.0)
        bad = (cs < 1.0) | ~jnp.isfinite(cs)
        out_ref[...] = jnp.where(bad, float(BASE_CHUNK), cs)


def kernel(node_features, edge_features, sequence_mask, W_node, b_node,
           W_e1, b_e1, W_e2, b_e2, W_imp, b_imp):
    edge_r = jnp.transpose(edge_features, (0, 1, 3, 2))
    wn_t = jnp.transpose(W_node)
    we2_t = jnp.transpose(W_e2)
    wimp_t = jnp.transpose(W_imp)
    bn = b_node.reshape(1, 32)
    be1 = b_e1.reshape(1, 64)
    be2 = b_e2.reshape(1, 32)
    bi = b_imp.reshape(1, 1)

    out = pl.pallas_call(
        _chunk_kernel,
        grid=(B, K),
        in_specs=[
            pl.BlockSpec((1, TS, DE, S), lambda b, k: (b, k, 0, 0)),
            pl.BlockSpec((1, TS, HIDDEN), lambda b, k: (b, k, 0)),
            pl.BlockSpec((B, S), lambda b, k: (0, 0)),
            pl.BlockSpec((32, HIDDEN), lambda b, k: (0, 0)),
            pl.BlockSpec((1, 32), lambda b, k: (0, 0)),
            pl.BlockSpec((1, 64), lambda b, k: (0, 0)),
            pl.BlockSpec((1, 64), lambda b, k: (0, 0)),
            pl.BlockSpec((32, 64), lambda b, k: (0, 0)),
            pl.BlockSpec((1, 32), lambda b, k: (0, 0)),
            pl.BlockSpec((1, 64), lambda b, k: (0, 0)),
            pl.BlockSpec((1, 1), lambda b, k: (0, 0)),
        ],
        out_specs=pl.BlockSpec((1, 1, 128), lambda b, k: (b, 0, 0)),
        out_shape=jax.ShapeDtypeStruct((B, 1, 128), jnp.float32),
        compiler_params=pltpu.CompilerParams(
            dimension_semantics=("parallel", "arbitrary")),
    )(edge_r, node_features, sequence_mask, wn_t, bn, W_e1, be1, we2_t,
      be2, wimp_t, bi)

    return (out[:, 0, 0], MAX_SEQ_LEN)
